# baseline (speedup 1.0000x reference)
import functools

import jax
import jax.numpy as jnp
from jax import lax
from jax.experimental import pallas as pl
from jax.experimental.pallas import tpu as pltpu
from jax.experimental.pallas import tpu_sc as plsc


def _gather_params_sc(gamma, beta, idx, B, C):
    @functools.partial(
        pl.kernel,
        out_type=(
            jax.ShapeDtypeStruct((B, C), jnp.float32),
            jax.ShapeDtypeStruct((B, C), jnp.float32),
        ),
        mesh=plsc.VectorSubcoreMesh(core_axis_name="c", subcore_axis_name="s"),
        scratch_types=[
            pltpu.VMEM((16,), jnp.int32),
            pltpu.VMEM((B, C), jnp.float32),
            pltpu.SemaphoreType.DMA,
        ],
    )
    def gather_kernel(gamma_hbm, beta_hbm, idx_hbm, g_out, t_out, idx_v, rows_v, sem):
        cid = lax.axis_index("c")
        sid = lax.axis_index("s")

        def gather_rows(table_hbm, dst_out):
            pltpu.sync_copy(idx_hbm, idx_v.at[pl.ds(0, B)])
            iv = idx_v[...]
            copies = [
                pltpu.make_async_copy(table_hbm.at[iv[b]], rows_v.at[b], sem)
                for b in range(B)
            ]
            for c in copies:
                c.start()
            for c in copies:
                c.wait()
            pltpu.sync_copy(rows_v, dst_out)

        @pl.when(jnp.logical_and(cid == 0, sid == 0))
        def _():
            gather_rows(gamma_hbm, g_out)

        @pl.when(jnp.logical_and(cid == 1, sid == 0))
        def _():
            gather_rows(beta_hbm, t_out)

    return gather_kernel(gamma, beta, idx)


def _affine_body(x_ref, g_ref, t_ref, o_ref):
    b = pl.program_id(0)
    g = g_ref[pl.ds(b, 1), :]
    t = t_ref[pl.ds(b, 1), :]
    o_ref[...] = x_ref[...] * g[0][None, None, None, :] + t[0][None, None, None, :]


def kernel(x, class_idx, gamma, beta):
    B, H, W, C = x.shape
    idx = class_idx.astype(jnp.int32)

    g_sel, t_sel = _gather_params_sc(gamma, beta, idx, B, C)

    TH = 112
    assert H % TH == 0
    out = pl.pallas_call(
        _affine_body,
        grid=(B, H // TH),
        in_specs=[
            pl.BlockSpec((1, TH, W, C), lambda b, h: (b, h, 0, 0)),
            pl.BlockSpec((B, C), lambda b, h: (0, 0)),
            pl.BlockSpec((B, C), lambda b, h: (0, 0)),
        ],
        out_specs=pl.BlockSpec((1, TH, W, C), lambda b, h: (b, h, 0, 0)),
        out_shape=jax.ShapeDtypeStruct((B, H, W, C), jnp.float32),
        compiler_params=pltpu.CompilerParams(
            dimension_semantics=("parallel", "arbitrary"),
        ),
    )(x, g_sel, t_sel)

    return out
